# baseline (speedup 1.0000x reference)
import functools

import jax
import jax.numpy as jnp
from jax import lax
from jax.experimental import pallas as pl
from jax.experimental.pallas import tpu as pltpu
from jax.experimental.pallas import tpu_sc as plsc

N = 10000
E = 320000
F = 128
H = 64
C = 16
G = 64

NC = 2
NS = 16
NW = NC * NS
EPW = E // NW
K = 128
NCHUNK = 80
NACC = 10240
ZSTR = 640
ZR = 160
NBUF = 2
RPT = 624
TAIL = N - NS * RPT

_POOL_PREC = lax.Precision.HIGHEST
_MM_PREC = lax.Precision.DEFAULT


def _dot(a, b, prec):
    return lax.dot_general(a, b, (((1,), (0,)), ((), ())),
                           precision=prec, preferred_element_type=jnp.float32)


def _pool(batch_col, h):
    oh = (batch_col == lax.broadcasted_iota(jnp.int32, (N, G), 1)
          ).astype(jnp.float32)
    return lax.dot_general(oh, h, (((0,), (0,)), ((), ())),
                           precision=_POOL_PREC,
                           preferred_element_type=jnp.float32)


def _make_agg(D):
    mesh = plsc.VectorSubcoreMesh(core_axis_name="c", subcore_axis_name="s",
                                  num_cores=NC, num_subcores=NS)

    @functools.partial(
        pl.kernel,
        out_type=jax.ShapeDtypeStruct((NC, N, D), jnp.float32),
        mesh=mesh,
        scratch_types=[
            pltpu.VMEM((NCHUNK, K), jnp.int32),
            pltpu.VMEM((NCHUNK, K), jnp.int32),
            pltpu.VMEM((NBUF, K, D), jnp.float32),
            pltpu.VMEM((ZR, D), jnp.float32),
            pltpu.VMEM_SHARED((NACC, D), jnp.float32),
            [pltpu.SemaphoreType.DMA] * NBUF,
        ],
        compiler_params=pltpu.CompilerParams(use_tc_tiling_on_sc=False),
    )
    def agg_kernel(y_hbm, src_hbm, dst_hbm, out_hbm,
                   src_v, dst_v, rows_v, zero_v, acc, gsem):
        c = lax.axis_index("c")
        s = lax.axis_index("s")
        wid = c * NS + s

        @pl.loop(0, ZR)
        def _(i):
            @pl.loop(0, D // 16)
            def _(j):
                zero_v[i, pl.ds(j * 16, 16)] = jnp.zeros((16,), jnp.float32)

        @pl.loop(0, ZSTR // ZR)
        def _(t):
            pltpu.sync_copy(zero_v, acc.at[pl.ds(s * ZSTR + t * ZR, ZR)])

        pltpu.sync_copy(src_hbm.at[wid], src_v)
        pltpu.sync_copy(dst_hbm.at[wid], dst_v)
        plsc.subcore_barrier()

        def start_gather(j, b):
            pltpu.async_copy(y_hbm.at[src_v.at[j]], rows_v.at[b], gsem[b])

        def wait_gather(j, b):
            pltpu.make_async_copy(y_hbm.at[src_v.at[j]], rows_v.at[b],
                                  gsem[b]).wait()

        def scatter(j, b):
            pltpu.sync_copy(rows_v.at[b], acc.at[dst_v.at[j]], add=True)

        start_gather(0, 0)

        @pl.loop(1, NCHUNK - 1, step=2)
        def _(j):
            start_gather(j, 1)
            wait_gather(j - 1, 0)
            scatter(j - 1, 0)
            start_gather(j + 1, 0)
            wait_gather(j, 1)
            scatter(j, 1)

        start_gather(NCHUNK - 1, 1)
        wait_gather(NCHUNK - 2, 0)
        scatter(NCHUNK - 2, 0)
        wait_gather(NCHUNK - 1, 1)
        scatter(NCHUNK - 1, 1)

        plsc.subcore_barrier()
        pltpu.sync_copy(acc.at[pl.ds(s * RPT, RPT)],
                        out_hbm.at[c, pl.ds(s * RPT, RPT)])

        @pl.when(s == NS - 1)
        def _():
            pltpu.sync_copy(acc.at[pl.ds(NS * RPT, TAIL)],
                            out_hbm.at[c, pl.ds(NS * RPT, TAIL)])

    return agg_kernel


_agg_cache = {}


def _agg(y, src, dst):
    d = y.shape[1]
    if d not in _agg_cache:
        _agg_cache[d] = _make_agg(d)
    return _agg_cache[d](y, src, dst)


def _agg_wide(x, src, dst):
    lo = _agg(x[:, :H], src, dst)
    hi = _agg(x[:, H:], src, dst)
    return lo, hi


def _bn(h, g, b):
    m = jnp.mean(h, axis=0, keepdims=True)
    v = jnp.mean(h * h, axis=0, keepdims=True) - m * m
    return (h - m) * lax.rsqrt(v + 1e-5) * g + b


def _layer_body(wide, x_ref, *refs):
    if wide:
        agglo_ref, agghi_ref, *refs = refs
        agg = jnp.concatenate([agglo_ref[0] + agglo_ref[1],
                               agghi_ref[0] + agghi_ref[1]], axis=1)
    else:
        agg_ref, *refs = refs
        agg = agg_ref[0] + agg_ref[1]
    (w1_ref, b1_ref, g1_ref, bt1_ref, w2_ref, b2_ref, bng_ref, bnb_ref,
     batch_ref, h_ref, pool_ref) = refs
    xin = x_ref[...] + agg
    s = _dot(xin, w1_ref[...], _MM_PREC) + b1_ref[...]
    r = jax.nn.relu(_bn(s, g1_ref[...], bt1_ref[...]))
    u = _dot(r, w2_ref[...], _MM_PREC) + b2_ref[...]
    h_ref[...] = jax.nn.relu(_bn(u, bng_ref[...], bnb_ref[...]))
    pool_ref[...] = _pool(batch_ref[...], x_ref[...])


def _make_layer(D, wide):
    return pl.pallas_call(
        functools.partial(_layer_body, wide),
        out_shape=[jax.ShapeDtypeStruct((N, H), jnp.float32),
                   jax.ShapeDtypeStruct((G, D), jnp.float32)],
        compiler_params=pltpu.CompilerParams(vmem_limit_bytes=63 * 1024 * 1024))


_layer1 = _make_layer(F, True)
_layer2 = _make_layer(H, False)


def _final_body(h2_ref, batch_ref, p0_ref, p1_ref, w0_ref, w1_ref, w2_ref,
                b_ref, out_ref):
    p2 = _pool(batch_ref[...], h2_ref[...])
    out_ref[...] = (_dot(p0_ref[...], w0_ref[...], _MM_PREC)
                    + _dot(p1_ref[...], w1_ref[...], _MM_PREC)
                    + _dot(p2, w2_ref[...], _MM_PREC)
                    + b_ref[...])


def _final(h2, batch_col, p0, p1, w0, w1, w2, b):
    return pl.pallas_call(
        _final_body,
        out_shape=jax.ShapeDtypeStruct((G, C), jnp.float32),
    )(h2, batch_col, p0, p1, w0, w1, w2, b)


def kernel(x, params, edge_index, batch):
    p = params
    pad = NCHUNK * K - EPW
    src = jnp.pad(edge_index[0].astype(jnp.int32).reshape(NW, EPW),
                  ((0, 0), (0, pad))).reshape(NW, NCHUNK, K)
    spare = (NACC - N) // NS - 1
    dummy = (N + (jnp.arange(NW, dtype=jnp.int32)[:, None] % NS) * spare
             + jnp.arange(pad, dtype=jnp.int32)[None, :] % spare)
    dst = jnp.concatenate(
        [edge_index[1].astype(jnp.int32).reshape(NW, EPW), dummy],
        axis=1).reshape(NW, NCHUNK, K)
    batch_col = batch.astype(jnp.int32).reshape(N, 1)

    def row(v):
        return v.reshape(1, -1)

    c1, c2 = p['conv1'], p['conv2']

    agg1_lo, agg1_hi = _agg_wide(x, src, dst)
    h1, pooled0 = _layer1(
        x, agg1_lo, agg1_hi, c1['W1'], row(c1['b1']), row(c1['g']),
        row(c1['bt']), c1['W2'], row(c1['b2']), row(p['bn1_g']),
        row(p['bn1_b']), batch_col)
    agg2 = _agg(h1, src, dst)
    h2, pooled1 = _layer2(
        h1, agg2, c2['W1'], row(c2['b1']), row(c2['g']), row(c2['bt']),
        c2['W2'], row(c2['b2']), row(p['bn2_g']), row(p['bn2_b']), batch_col)

    bias = row(p['lin0_b'] + p['lin1_b'] + p['lin2_b'])
    out = _final(h2, batch_col, pooled0, pooled1,
                 p['lin0_W'], p['lin1_W'], p['lin2_W'], bias)
    return (x, h1, h2, out)


# speedup vs baseline: 2.0900x; 2.0900x over previous
import functools

import jax
import jax.numpy as jnp
from jax import lax
from jax.experimental import pallas as pl
from jax.experimental.pallas import tpu as pltpu
from jax.experimental.pallas import tpu_sc as plsc

N = 10000
E = 320000
F = 128
H = 64
C = 16
G = 64

NC = 2
NS = 16
NW = NC * NS
EPW = E // NW
K = 80
NCHUNK = EPW // K
NACC = 10240
ZSTR = 640
ZR = 160
NBUF = 2
RPT = 624
TAIL = N - NS * RPT

_POOL_PREC = lax.Precision.HIGHEST
_MM_PREC = lax.Precision.DEFAULT


def _dot(a, b, prec):
    return lax.dot_general(a, b, (((1,), (0,)), ((), ())),
                           precision=prec, preferred_element_type=jnp.float32)


def _pool(batch_col, h):
    oh = (batch_col == lax.broadcasted_iota(jnp.int32, (N, G), 1)
          ).astype(jnp.float32)
    return lax.dot_general(oh, h, (((0,), (0,)), ((), ())),
                           precision=_POOL_PREC,
                           preferred_element_type=jnp.float32)


def _make_agg(D):
    mesh = plsc.VectorSubcoreMesh(core_axis_name="c", subcore_axis_name="s",
                                  num_cores=NC, num_subcores=NS)

    @functools.partial(
        pl.kernel,
        out_type=jax.ShapeDtypeStruct((NC, N, D), jnp.float32),
        mesh=mesh,
        scratch_types=[
            pltpu.VMEM((NCHUNK, K), jnp.int32),
            pltpu.VMEM((NCHUNK, K), jnp.int32),
            pltpu.VMEM((NBUF, K, D), jnp.float32),
            pltpu.VMEM((ZR, D), jnp.float32),
            pltpu.VMEM_SHARED((NACC, D), jnp.float32),
            [pltpu.SemaphoreType.DMA] * NBUF,
        ],
        compiler_params=pltpu.CompilerParams(use_tc_tiling_on_sc=False),
    )
    def agg_kernel(y_hbm, src_hbm, dst_hbm, out_hbm,
                   src_v, dst_v, rows_v, zero_v, acc, gsem):
        c = lax.axis_index("c")
        s = lax.axis_index("s")
        wid = c * NS + s

        @pl.loop(0, ZR)
        def _(i):
            @pl.loop(0, D // 16)
            def _(j):
                zero_v[i, pl.ds(j * 16, 16)] = jnp.zeros((16,), jnp.float32)

        @pl.loop(0, ZSTR // ZR)
        def _(t):
            pltpu.sync_copy(zero_v, acc.at[pl.ds(s * ZSTR + t * ZR, ZR)])

        pltpu.sync_copy(src_hbm.at[wid], src_v)
        pltpu.sync_copy(dst_hbm.at[wid], dst_v)
        plsc.subcore_barrier()

        def start_gather(j, b):
            pltpu.async_copy(y_hbm.at[src_v.at[j]], rows_v.at[b], gsem[b])

        def wait_gather(j, b):
            pltpu.make_async_copy(y_hbm.at[src_v.at[j]], rows_v.at[b],
                                  gsem[b]).wait()

        def scatter(j, b):
            pltpu.sync_copy(rows_v.at[b], acc.at[dst_v.at[j]], add=True)

        start_gather(0, 0)

        @pl.loop(1, NCHUNK, step=2)
        def _(j):
            start_gather(j, 1)
            wait_gather(j - 1, 0)
            scatter(j - 1, 0)
            start_gather(j + 1, 0)
            wait_gather(j, 1)
            scatter(j, 1)

        wait_gather(NCHUNK - 1, 0)
        scatter(NCHUNK - 1, 0)

        plsc.subcore_barrier()
        pltpu.sync_copy(acc.at[pl.ds(s * RPT, RPT)],
                        out_hbm.at[c, pl.ds(s * RPT, RPT)])

        @pl.when(s == NS - 1)
        def _():
            pltpu.sync_copy(acc.at[pl.ds(NS * RPT, TAIL)],
                            out_hbm.at[c, pl.ds(NS * RPT, TAIL)])

    return agg_kernel


_agg_cache = {}


def _agg(y, src, dst):
    d = y.shape[1]
    if d not in _agg_cache:
        _agg_cache[d] = _make_agg(d)
    return _agg_cache[d](y, src, dst)


def _agg_wide(x, src, dst):
    lo = _agg(x[:, :H], src, dst)
    hi = _agg(x[:, H:], src, dst)
    return lo, hi


def _bn(h, g, b):
    m = jnp.mean(h, axis=0, keepdims=True)
    v = jnp.mean(h * h, axis=0, keepdims=True) - m * m
    return (h - m) * lax.rsqrt(v + 1e-5) * g + b


def _layer_body(wide, x_ref, *refs):
    if wide:
        agglo_ref, agghi_ref, *refs = refs
        agg = jnp.concatenate([agglo_ref[0] + agglo_ref[1],
                               agghi_ref[0] + agghi_ref[1]], axis=1)
    else:
        agg_ref, *refs = refs
        agg = agg_ref[0] + agg_ref[1]
    (w1_ref, b1_ref, g1_ref, bt1_ref, w2_ref, b2_ref, bng_ref, bnb_ref,
     batch_ref, h_ref, pool_ref) = refs
    xin = x_ref[...] + agg
    s = _dot(xin, w1_ref[...], _MM_PREC) + b1_ref[...]
    r = jax.nn.relu(_bn(s, g1_ref[...], bt1_ref[...]))
    u = _dot(r, w2_ref[...], _MM_PREC) + b2_ref[...]
    h_ref[...] = jax.nn.relu(_bn(u, bng_ref[...], bnb_ref[...]))
    pool_ref[...] = _pool(batch_ref[...], x_ref[...])


def _make_layer(D, wide):
    return pl.pallas_call(
        functools.partial(_layer_body, wide),
        out_shape=[jax.ShapeDtypeStruct((N, H), jnp.float32),
                   jax.ShapeDtypeStruct((G, D), jnp.float32)],
        compiler_params=pltpu.CompilerParams(vmem_limit_bytes=63 * 1024 * 1024))


_layer1 = _make_layer(F, True)
_layer2 = _make_layer(H, False)


def _final_body(h2_ref, batch_ref, p0_ref, p1_ref, w0_ref, w1_ref, w2_ref,
                b_ref, out_ref):
    p2 = _pool(batch_ref[...], h2_ref[...])
    out_ref[...] = (_dot(p0_ref[...], w0_ref[...], _MM_PREC)
                    + _dot(p1_ref[...], w1_ref[...], _MM_PREC)
                    + _dot(p2, w2_ref[...], _MM_PREC)
                    + b_ref[...])


def _final(h2, batch_col, p0, p1, w0, w1, w2, b):
    return pl.pallas_call(
        _final_body,
        out_shape=jax.ShapeDtypeStruct((G, C), jnp.float32),
    )(h2, batch_col, p0, p1, w0, w1, w2, b)


def kernel(x, params, edge_index, batch):
    p = params
    src = edge_index[0].astype(jnp.int32).reshape(NW, NCHUNK, K)
    dst = edge_index[1].astype(jnp.int32).reshape(NW, NCHUNK, K)
    batch_col = batch.astype(jnp.int32).reshape(N, 1)

    def row(v):
        return v.reshape(1, -1)

    c1, c2 = p['conv1'], p['conv2']

    agg1_lo, agg1_hi = _agg_wide(x, src, dst)
    h1, pooled0 = _layer1(
        x, agg1_lo, agg1_hi, c1['W1'], row(c1['b1']), row(c1['g']),
        row(c1['bt']), c1['W2'], row(c1['b2']), row(p['bn1_g']),
        row(p['bn1_b']), batch_col)
    agg2 = _agg(h1, src, dst)
    h2, pooled1 = _layer2(
        h1, agg2, c2['W1'], row(c2['b1']), row(c2['g']), row(c2['bt']),
        c2['W2'], row(c2['b2']), row(p['bn2_g']), row(p['bn2_b']), batch_col)

    bias = row(p['lin0_b'] + p['lin1_b'] + p['lin2_b'])
    out = _final(h2, batch_col, pooled0, pooled1,
                 p['lin0_W'], p['lin1_W'], p['lin2_W'], bias)
    return (x, h1, h2, out)


# speedup vs baseline: 2.5863x; 1.2375x over previous
import functools

import jax
import jax.numpy as jnp
from jax import lax
from jax.experimental import pallas as pl
from jax.experimental.pallas import tpu as pltpu
from jax.experimental.pallas import tpu_sc as plsc

N = 10000
E = 320000
F = 128
H = 64
C = 16
G = 64

NC = 2
NS = 16
NW = NC * NS
EPW = E // NW
K = 80
NCHUNK = EPW // K
NACC = 10240
ZSTR = 640
ZR = 160
NBUF = 4
RPT = 624
TAIL = N - NS * RPT

_POOL_PREC = lax.Precision.HIGHEST
_MM_PREC = lax.Precision.DEFAULT


def _dot(a, b, prec):
    return lax.dot_general(a, b, (((1,), (0,)), ((), ())),
                           precision=prec, preferred_element_type=jnp.float32)


def _pool(batch_col, h):
    oh = (batch_col == lax.broadcasted_iota(jnp.int32, (N, G), 1)
          ).astype(jnp.float32)
    return lax.dot_general(oh, h, (((0,), (0,)), ((), ())),
                           precision=_POOL_PREC,
                           preferred_element_type=jnp.float32)


def _make_agg(D):
    mesh = plsc.VectorSubcoreMesh(core_axis_name="c", subcore_axis_name="s",
                                  num_cores=NC, num_subcores=NS)

    @functools.partial(
        pl.kernel,
        out_type=jax.ShapeDtypeStruct((NC, N, D), jnp.float32),
        mesh=mesh,
        scratch_types=[
            pltpu.VMEM((NCHUNK, K), jnp.int32),
            pltpu.VMEM((NCHUNK, K), jnp.int32),
            pltpu.VMEM((NBUF, K, D), jnp.float32),
            pltpu.VMEM((ZR, D), jnp.float32),
            pltpu.VMEM_SHARED((NACC, D), jnp.float32),
            [pltpu.SemaphoreType.DMA] * NBUF,
            [pltpu.SemaphoreType.DMA] * NBUF,
        ],
        compiler_params=pltpu.CompilerParams(use_tc_tiling_on_sc=False),
    )
    def agg_kernel(y_hbm, src_hbm, dst_hbm, out_hbm,
                   src_v, dst_v, rows_v, zero_v, acc, gsem, ssem):
        c = lax.axis_index("c")
        s = lax.axis_index("s")
        wid = c * NS + s

        @pl.loop(0, ZR)
        def _(i):
            @pl.loop(0, D // 16)
            def _(j):
                zero_v[i, pl.ds(j * 16, 16)] = jnp.zeros((16,), jnp.float32)

        @pl.loop(0, ZSTR // ZR)
        def _(t):
            pltpu.sync_copy(zero_v, acc.at[pl.ds(s * ZSTR + t * ZR, ZR)])

        pltpu.sync_copy(src_hbm.at[wid], src_v)
        pltpu.sync_copy(dst_hbm.at[wid], dst_v)
        plsc.subcore_barrier()

        def start_gather(j, b):
            pltpu.async_copy(y_hbm.at[src_v.at[j]], rows_v.at[b], gsem[b])

        def wait_gather(j, b):
            pltpu.make_async_copy(y_hbm.at[src_v.at[j]], rows_v.at[b],
                                  gsem[b]).wait()

        def start_scatter(j, b):
            pltpu.async_copy(rows_v.at[b], acc.at[dst_v.at[j]], ssem[b],
                             add=True)

        def wait_scatter(j, b):
            pltpu.make_async_copy(rows_v.at[b], acc.at[dst_v.at[j]],
                                  ssem[b]).wait()

        start_gather(0, 0)
        start_gather(1, 1)
        start_gather(2, 2)
        wait_gather(0, 0)
        start_scatter(0, 0)
        start_gather(3, 3)

        @pl.loop(1, NCHUNK - 4, step=4)
        def _(jb):
            for t in range(4):
                b = (1 + t) % NBUF
                bn = (b + 3) % NBUF
                j = jb + t
                wait_gather(j, b)
                start_scatter(j, b)
                wait_scatter(j - 1, bn)
                start_gather(j + 3, bn)

        j = NCHUNK - 4
        wait_gather(j, j % NBUF)
        start_scatter(j, j % NBUF)
        wait_scatter(j - 1, (j + 3) % NBUF)
        start_gather(j + 3, (j + 3) % NBUF)
        for j in range(NCHUNK - 3, NCHUNK):
            wait_gather(j, j % NBUF)
            start_scatter(j, j % NBUF)
            wait_scatter(j - 1, (j - 1) % NBUF)
        wait_scatter(NCHUNK - 1, (NCHUNK - 1) % NBUF)

        plsc.subcore_barrier()
        pltpu.sync_copy(acc.at[pl.ds(s * RPT, RPT)],
                        out_hbm.at[c, pl.ds(s * RPT, RPT)])

        @pl.when(s == NS - 1)
        def _():
            pltpu.sync_copy(acc.at[pl.ds(NS * RPT, TAIL)],
                            out_hbm.at[c, pl.ds(NS * RPT, TAIL)])

    return agg_kernel


_agg_cache = {}


def _agg(y, src, dst):
    d = y.shape[1]
    if d not in _agg_cache:
        _agg_cache[d] = _make_agg(d)
    return _agg_cache[d](y, src, dst)


def _agg_wide(x, src, dst):
    lo = _agg(x[:, :H], src, dst)
    hi = _agg(x[:, H:], src, dst)
    return lo, hi


def _bn(h, g, b):
    m = jnp.mean(h, axis=0, keepdims=True)
    v = jnp.mean(h * h, axis=0, keepdims=True) - m * m
    return (h - m) * lax.rsqrt(v + 1e-5) * g + b


def _layer_body(wide, x_ref, *refs):
    if wide:
        agglo_ref, agghi_ref, *refs = refs
        agg = jnp.concatenate([agglo_ref[0] + agglo_ref[1],
                               agghi_ref[0] + agghi_ref[1]], axis=1)
    else:
        agg_ref, *refs = refs
        agg = agg_ref[0] + agg_ref[1]
    (w1_ref, b1_ref, g1_ref, bt1_ref, w2_ref, b2_ref, bng_ref, bnb_ref,
     batch_ref, h_ref, pool_ref) = refs
    xin = x_ref[...] + agg
    s = _dot(xin, w1_ref[...], _MM_PREC) + b1_ref[...]
    r = jax.nn.relu(_bn(s, g1_ref[...], bt1_ref[...]))
    u = _dot(r, w2_ref[...], _MM_PREC) + b2_ref[...]
    h_ref[...] = jax.nn.relu(_bn(u, bng_ref[...], bnb_ref[...]))
    pool_ref[...] = _pool(batch_ref[...], x_ref[...])


def _make_layer(D, wide):
    return pl.pallas_call(
        functools.partial(_layer_body, wide),
        out_shape=[jax.ShapeDtypeStruct((N, H), jnp.float32),
                   jax.ShapeDtypeStruct((G, D), jnp.float32)],
        compiler_params=pltpu.CompilerParams(vmem_limit_bytes=63 * 1024 * 1024))


_layer1 = _make_layer(F, True)
_layer2 = _make_layer(H, False)


def _final_body(h2_ref, batch_ref, p0_ref, p1_ref, w0_ref, w1_ref, w2_ref,
                b_ref, out_ref):
    p2 = _pool(batch_ref[...], h2_ref[...])
    out_ref[...] = (_dot(p0_ref[...], w0_ref[...], _MM_PREC)
                    + _dot(p1_ref[...], w1_ref[...], _MM_PREC)
                    + _dot(p2, w2_ref[...], _MM_PREC)
                    + b_ref[...])


def _final(h2, batch_col, p0, p1, w0, w1, w2, b):
    return pl.pallas_call(
        _final_body,
        out_shape=jax.ShapeDtypeStruct((G, C), jnp.float32),
    )(h2, batch_col, p0, p1, w0, w1, w2, b)


def kernel(x, params, edge_index, batch):
    p = params
    src = edge_index[0].astype(jnp.int32).reshape(NW, NCHUNK, K)
    dst = edge_index[1].astype(jnp.int32).reshape(NW, NCHUNK, K)
    batch_col = batch.astype(jnp.int32).reshape(N, 1)

    def row(v):
        return v.reshape(1, -1)

    c1, c2 = p['conv1'], p['conv2']

    agg1_lo, agg1_hi = _agg_wide(x, src, dst)
    h1, pooled0 = _layer1(
        x, agg1_lo, agg1_hi, c1['W1'], row(c1['b1']), row(c1['g']),
        row(c1['bt']), c1['W2'], row(c1['b2']), row(p['bn1_g']),
        row(p['bn1_b']), batch_col)
    agg2 = _agg(h1, src, dst)
    h2, pooled1 = _layer2(
        h1, agg2, c2['W1'], row(c2['b1']), row(c2['g']), row(c2['bt']),
        c2['W2'], row(c2['b2']), row(p['bn2_g']), row(p['bn2_b']), batch_col)

    bias = row(p['lin0_b'] + p['lin1_b'] + p['lin2_b'])
    out = _final(h2, batch_col, pooled0, pooled1,
                 p['lin0_W'], p['lin1_W'], p['lin2_W'], bias)
    return (x, h1, h2, out)


# speedup vs baseline: 2.5985x; 1.0047x over previous
import functools

import jax
import jax.numpy as jnp
from jax import lax
from jax.experimental import pallas as pl
from jax.experimental.pallas import tpu as pltpu
from jax.experimental.pallas import tpu_sc as plsc

N = 10000
E = 320000
F = 128
H = 64
C = 16
G = 64

NC = 2
NS = 16
NW = NC * NS
EPW = E // NW
K = 80
NCHUNK = EPW // K
NACC = 10240
ZSTR = 640
ZR = 160
NBUF = 4
RPT = 624
TAIL = N - NS * RPT

_POOL_PREC = lax.Precision.HIGHEST
_MM_PREC = lax.Precision.DEFAULT


def _dot(a, b, prec):
    return lax.dot_general(a, b, (((1,), (0,)), ((), ())),
                           precision=prec, preferred_element_type=jnp.float32)


def _pool(batch_col, h):
    oh = (batch_col == lax.broadcasted_iota(jnp.int32, (N, G), 1)
          ).astype(jnp.float32)
    return lax.dot_general(oh, h, (((0,), (0,)), ((), ())),
                           precision=_POOL_PREC,
                           preferred_element_type=jnp.float32)


def _make_agg(D, nparts):
    mesh = plsc.VectorSubcoreMesh(core_axis_name="c", subcore_axis_name="s",
                                  num_cores=NC, num_subcores=NS)

    @functools.partial(
        pl.kernel,
        out_type=[jax.ShapeDtypeStruct((NC, N, D), jnp.float32)] * nparts,
        mesh=mesh,
        scratch_types=[
            pltpu.VMEM((NCHUNK, K), jnp.int32),
            pltpu.VMEM((NCHUNK, K), jnp.int32),
            pltpu.VMEM((NBUF, K, D), jnp.float32),
            pltpu.VMEM((ZR, D), jnp.float32),
            pltpu.VMEM_SHARED((NACC, D), jnp.float32),
            [pltpu.SemaphoreType.DMA] * NBUF,
            [pltpu.SemaphoreType.DMA] * NBUF,
        ],
        compiler_params=pltpu.CompilerParams(use_tc_tiling_on_sc=False),
    )
    def agg_kernel(*refs):
        y_hbms, refs = refs[:nparts], refs[nparts:]
        src_hbm, dst_hbm = refs[0], refs[1]
        out_hbms = refs[2:2 + nparts]
        src_v, dst_v, rows_v, zero_v, acc, gsem, ssem = refs[2 + nparts:]
        c = lax.axis_index("c")
        s = lax.axis_index("s")
        wid = c * NS + s

        @pl.loop(0, ZR)
        def _(i):
            @pl.loop(0, D // 16)
            def _(j):
                zero_v[i, pl.ds(j * 16, 16)] = jnp.zeros((16,), jnp.float32)

        pltpu.sync_copy(src_hbm.at[wid], src_v)
        pltpu.sync_copy(dst_hbm.at[wid], dst_v)

        def zero_acc():
            @pl.loop(0, ZSTR // ZR)
            def _(t):
                pltpu.sync_copy(zero_v, acc.at[pl.ds(s * ZSTR + t * ZR, ZR)])

        def pipeline(y_hbm):
            def start_gather(j, b):
                pltpu.async_copy(y_hbm.at[src_v.at[j]], rows_v.at[b], gsem[b])

            def wait_gather(j, b):
                pltpu.make_async_copy(y_hbm.at[src_v.at[j]], rows_v.at[b],
                                      gsem[b]).wait()

            def start_scatter(j, b):
                pltpu.async_copy(rows_v.at[b], acc.at[dst_v.at[j]], ssem[b],
                                 add=True)

            def wait_scatter(j, b):
                pltpu.make_async_copy(rows_v.at[b], acc.at[dst_v.at[j]],
                                      ssem[b]).wait()

            start_gather(0, 0)
            start_gather(1, 1)
            start_gather(2, 2)
            wait_gather(0, 0)
            start_scatter(0, 0)
            start_gather(3, 3)

            @pl.loop(1, NCHUNK - 4, step=4)
            def _(jb):
                for t in range(4):
                    b = (1 + t) % NBUF
                    bn = (b + 3) % NBUF
                    j = jb + t
                    wait_gather(j, b)
                    start_scatter(j, b)
                    wait_scatter(j - 1, bn)
                    start_gather(j + 3, bn)

            j = NCHUNK - 4
            wait_gather(j, j % NBUF)
            start_scatter(j, j % NBUF)
            wait_scatter(j - 1, (j + 3) % NBUF)
            start_gather(j + 3, (j + 3) % NBUF)
            for j in range(NCHUNK - 3, NCHUNK):
                wait_gather(j, j % NBUF)
                start_scatter(j, j % NBUF)
                wait_scatter(j - 1, (j - 1) % NBUF)
            wait_scatter(NCHUNK - 1, (NCHUNK - 1) % NBUF)

        def drain(out_hbm):
            pltpu.sync_copy(acc.at[pl.ds(s * RPT, RPT)],
                            out_hbm.at[c, pl.ds(s * RPT, RPT)])

            @pl.when(s == NS - 1)
            def _():
                pltpu.sync_copy(acc.at[pl.ds(NS * RPT, TAIL)],
                                out_hbm.at[c, pl.ds(NS * RPT, TAIL)])

        for part in range(nparts):
            zero_acc()
            plsc.subcore_barrier()
            pipeline(y_hbms[part])
            plsc.subcore_barrier()
            drain(out_hbms[part])
            if part + 1 < nparts:
                plsc.subcore_barrier()

    return agg_kernel


_agg_cache = {}


def _agg(y, src, dst):
    if 1 not in _agg_cache:
        _agg_cache[1] = _make_agg(H, 1)
    return _agg_cache[1](y, src, dst)[0]


def _agg_wide(x, src, dst):
    if 2 not in _agg_cache:
        _agg_cache[2] = _make_agg(H, 2)
    return _agg_cache[2](x[:, :H], x[:, H:], src, dst)


def _bn(h, g, b):
    m = jnp.mean(h, axis=0, keepdims=True)
    v = jnp.mean(h * h, axis=0, keepdims=True) - m * m
    return (h - m) * lax.rsqrt(v + 1e-5) * g + b


def _layer_body(wide, x_ref, *refs):
    if wide:
        agglo_ref, agghi_ref, *refs = refs
        agg = jnp.concatenate([agglo_ref[0] + agglo_ref[1],
                               agghi_ref[0] + agghi_ref[1]], axis=1)
    else:
        agg_ref, *refs = refs
        agg = agg_ref[0] + agg_ref[1]
    (w1_ref, b1_ref, g1_ref, bt1_ref, w2_ref, b2_ref, bng_ref, bnb_ref,
     batch_ref, h_ref, pool_ref) = refs
    xin = x_ref[...] + agg
    s = _dot(xin, w1_ref[...], _MM_PREC) + b1_ref[...]
    r = jax.nn.relu(_bn(s, g1_ref[...], bt1_ref[...]))
    u = _dot(r, w2_ref[...], _MM_PREC) + b2_ref[...]
    h_ref[...] = jax.nn.relu(_bn(u, bng_ref[...], bnb_ref[...]))
    pool_ref[...] = _pool(batch_ref[...], x_ref[...])


def _make_layer(D, wide):
    return pl.pallas_call(
        functools.partial(_layer_body, wide),
        out_shape=[jax.ShapeDtypeStruct((N, H), jnp.float32),
                   jax.ShapeDtypeStruct((G, D), jnp.float32)],
        compiler_params=pltpu.CompilerParams(vmem_limit_bytes=63 * 1024 * 1024))


_layer1 = _make_layer(F, True)
_layer2 = _make_layer(H, False)


def _final_body(h2_ref, batch_ref, p0_ref, p1_ref, w0_ref, w1_ref, w2_ref,
                b_ref, out_ref):
    p2 = _pool(batch_ref[...], h2_ref[...])
    out_ref[...] = (_dot(p0_ref[...], w0_ref[...], _MM_PREC)
                    + _dot(p1_ref[...], w1_ref[...], _MM_PREC)
                    + _dot(p2, w2_ref[...], _MM_PREC)
                    + b_ref[...])


def _final(h2, batch_col, p0, p1, w0, w1, w2, b):
    return pl.pallas_call(
        _final_body,
        out_shape=jax.ShapeDtypeStruct((G, C), jnp.float32),
    )(h2, batch_col, p0, p1, w0, w1, w2, b)


def kernel(x, params, edge_index, batch):
    p = params
    src = edge_index[0].astype(jnp.int32).reshape(NW, NCHUNK, K)
    dst = edge_index[1].astype(jnp.int32).reshape(NW, NCHUNK, K)
    batch_col = batch.astype(jnp.int32).reshape(N, 1)

    def row(v):
        return v.reshape(1, -1)

    c1, c2 = p['conv1'], p['conv2']

    agg1_lo, agg1_hi = _agg_wide(x, src, dst)
    h1, pooled0 = _layer1(
        x, agg1_lo, agg1_hi, c1['W1'], row(c1['b1']), row(c1['g']),
        row(c1['bt']), c1['W2'], row(c1['b2']), row(p['bn1_g']),
        row(p['bn1_b']), batch_col)
    agg2 = _agg(h1, src, dst)
    h2, pooled1 = _layer2(
        h1, agg2, c2['W1'], row(c2['b1']), row(c2['g']), row(c2['bt']),
        c2['W2'], row(c2['b2']), row(p['bn2_g']), row(p['bn2_b']), batch_col)

    bias = row(p['lin0_b'] + p['lin1_b'] + p['lin2_b'])
    out = _final(h2, batch_col, pooled0, pooled1,
                 p['lin0_W'], p['lin1_W'], p['lin2_W'], bias)
    return (x, h1, h2, out)
